# baseline (speedup 1.0000x reference)
import functools

import jax
import jax.numpy as jnp
from jax import lax
from jax.experimental import pallas as pl
from jax.experimental.pallas import tpu as pltpu
from jax.experimental.pallas import tpu_sc as plsc

NODES = 10000
IN_DIM = 128
HID_DIM = 256
OUT_DIM = 128
GRAPHS = 16

NC = 2
NS = 16
NW = NC * NS
CHUNK = 128
CH = 80
PH = 2
CHP = CH // PH
ACH = 32
AP0 = 5
E0 = NS * AP0 * ACH * CHUNK
EPAD = NW * CH * CHUNK
NPAD = NODES + 112
RPT = NPAD // NS
RB = 1000
GRID = NODES // RB


def _sc_mesh():
    return plsc.VectorSubcoreMesh(core_axis_name="c", subcore_axis_name="s")


def _deg_call(dst_p, ones128, zerosf):
    @functools.partial(
        pl.kernel,
        out_type=jax.ShapeDtypeStruct((NC, NPAD, IN_DIM), jnp.float32),
        mesh=_sc_mesh(),
        scratch_types=[
            pltpu.VMEM((PH, CHP, CHUNK), jnp.int32),
            pltpu.VMEM((CHUNK, IN_DIM), jnp.float32),
            pltpu.VMEM_SHARED((NPAD, IN_DIM), jnp.float32),
        ],
    )
    def k(dst_hbm, ones_hbm, zeros_hbm, out_hbm, dst_v, ones_v, acc):
        cid = lax.axis_index("c")
        sid = lax.axis_index("s")
        wid = cid * NS + sid

        @pl.when(sid == 0)
        def _():
            pltpu.sync_copy(zeros_hbm, acc)

        pltpu.sync_copy(dst_hbm.at[wid], dst_v)
        pltpu.sync_copy(ones_hbm, ones_v)
        plsc.subcore_barrier()

        def body(p, c):
            def inner(j, c2):
                pltpu.sync_copy(ones_v, acc.at[dst_v.at[p, j]], add=True)
                return c2

            return lax.fori_loop(0, CHP, inner, c)

        lax.fori_loop(0, PH, body, 0)
        plsc.subcore_barrier()
        pltpu.sync_copy(
            acc.at[pl.ds(sid * RPT, RPT)],
            out_hbm.at[cid, pl.ds(sid * RPT, RPT)],
        )

    return k(dst_p, ones128, zerosf)


def _agg_call(y, srcA, dstA, zerosf):
    @functools.partial(
        pl.kernel,
        out_type=jax.ShapeDtypeStruct((NC, NPAD, IN_DIM), jnp.float32),
        mesh=_sc_mesh(),
        scratch_types=[
            pltpu.VMEM((ACH, CHUNK), jnp.int32),
            pltpu.VMEM((ACH, CHUNK), jnp.int32),
            pltpu.VMEM((CHUNK, IN_DIM), jnp.float32),
            pltpu.VMEM((CHUNK, IN_DIM), jnp.float32),
            pltpu.SemaphoreType.DMA,
            pltpu.SemaphoreType.DMA,
            pltpu.VMEM_SHARED((NPAD, IN_DIM), jnp.float32),
        ],
    )
    def k(y_hbm, srcA_hbm, dstA_hbm, zeros_hbm, out_hbm,
          src_v, dst_v, rows0, rows1, sem0, sem1, acc):
        cid = lax.axis_index("c")
        sid = lax.axis_index("s")

        @pl.when(sid == 0)
        def _():
            pltpu.sync_copy(zeros_hbm, acc)

        plsc.subcore_barrier()

        def run(src_hbm, dst_hbm, nph):
            def phase(p, c):
                pltpu.sync_copy(src_hbm.at[sid, p], src_v)
                pltpu.sync_copy(dst_hbm.at[sid, p], dst_v)
                pltpu.async_copy(y_hbm.at[src_v.at[0]], rows0, sem0)

                def body(t, c2):
                    j0 = 2 * t
                    j1 = j0 + 1
                    pltpu.async_copy(y_hbm.at[src_v.at[j1]], rows1, sem1)
                    pltpu.make_async_copy(y_hbm.at[src_v.at[j0]], rows0, sem0).wait()
                    pltpu.sync_copy(rows0, acc.at[dst_v.at[j0]], add=True)

                    @pl.when(j0 + 2 < ACH)
                    def _():
                        pltpu.async_copy(y_hbm.at[src_v.at[j0 + 2]], rows0, sem0)

                    pltpu.make_async_copy(y_hbm.at[src_v.at[j1]], rows1, sem1).wait()
                    pltpu.sync_copy(rows1, acc.at[dst_v.at[j1]], add=True)
                    return c2

                return lax.fori_loop(0, ACH // 2, body, c)

            lax.fori_loop(0, nph, phase, 0)

        @pl.when(cid == 0)
        def _():
            run(srcA_hbm, dstA_hbm, AP0)


        plsc.subcore_barrier()
        pltpu.sync_copy(
            acc.at[pl.ds(sid * RPT, RPT)],
            out_hbm.at[cid, pl.ds(sid * RPT, RPT)],
        )

    return k(y, srcA, dstA, zerosf)


def _dinv_block(degp_block):
    deg = degp_block[0, :, :1] + degp_block[1, :, :1] + 1.0
    return lax.rsqrt(deg)


def _scale_call(x, degp):
    def body(x_ref, degp_ref, y_ref):
        dinv = _dinv_block(degp_ref[...])
        y_ref[...] = x_ref[...] * dinv

    return pl.pallas_call(
        body,
        grid=(GRID,),
        in_specs=[
            pl.BlockSpec((RB, IN_DIM), lambda i: (i, 0)),
            pl.BlockSpec((NC, RB, IN_DIM), lambda i: (0, i, 0)),
        ],
        out_specs=pl.BlockSpec((RB, IN_DIM), lambda i: (i, 0)),
        out_shape=jax.ShapeDtypeStruct((NODES, IN_DIM), jnp.float32),
    )(x, degp)


def _mid_call(p, y0, degp, W1, b1, W2):
    def body(p_ref, y0_ref, degp_ref, w1_ref, b1_ref, w2_ref, y1_ref):
        dinv = _dinv_block(degp_ref[...])
        pv = p_ref[...]
        t0 = dinv * (pv[0] + pv[1] + y0_ref[...])
        h = jnp.maximum(
            jnp.dot(t0, w1_ref[...], preferred_element_type=jnp.float32)
            + b1_ref[...],
            0.0,
        )
        z = jnp.dot(h, w2_ref[...], preferred_element_type=jnp.float32)
        y1_ref[...] = dinv * z

    return pl.pallas_call(
        body,
        grid=(GRID,),
        in_specs=[
            pl.BlockSpec((NC, RB, IN_DIM), lambda i: (0, i, 0)),
            pl.BlockSpec((RB, IN_DIM), lambda i: (i, 0)),
            pl.BlockSpec((NC, RB, IN_DIM), lambda i: (0, i, 0)),
            pl.BlockSpec((IN_DIM, HID_DIM), lambda i: (0, 0)),
            pl.BlockSpec((1, HID_DIM), lambda i: (0, 0)),
            pl.BlockSpec((HID_DIM, OUT_DIM), lambda i: (0, 0)),
        ],
        out_specs=pl.BlockSpec((RB, OUT_DIM), lambda i: (i, 0)),
        out_shape=jax.ShapeDtypeStruct((NODES, OUT_DIM), jnp.float32),
    )(p, y0, degp, W1, b1, W2)


def _final_call(q, y1, degp, b2, batch3):
    def body(q_ref, y1_ref, degp_ref, b2_ref, batch_ref, out_ref, sums, cnts):
        i = pl.program_id(0)

        @pl.when(i == 0)
        def _():
            sums[...] = jnp.zeros_like(sums)
            cnts[...] = jnp.zeros_like(cnts)

        dinv = _dinv_block(degp_ref[...])
        qv = q_ref[...]
        t1 = dinv * (qv[0] + qv[1] + y1_ref[...])
        h2 = jnp.maximum(t1 + b2_ref[...], 0.0)
        b = batch_ref[...].reshape(1, RB)
        iota = lax.broadcasted_iota(jnp.int32, (GRAPHS, RB), 0)
        onehot = (jnp.broadcast_to(b, (GRAPHS, RB)) == iota).astype(jnp.float32)
        sums[...] += jnp.dot(onehot, h2, preferred_element_type=jnp.float32)
        cnts[...] += jnp.broadcast_to(
            jnp.sum(onehot, axis=1, keepdims=True), (GRAPHS, OUT_DIM)
        )

        @pl.when(i == GRID - 1)
        def _():
            out_ref[...] = sums[...] / jnp.maximum(cnts[...], 1.0)

    return pl.pallas_call(
        body,
        grid=(GRID,),
        in_specs=[
            pl.BlockSpec((NC, RB, IN_DIM), lambda i: (0, i, 0)),
            pl.BlockSpec((RB, OUT_DIM), lambda i: (i, 0)),
            pl.BlockSpec((NC, RB, IN_DIM), lambda i: (0, i, 0)),
            pl.BlockSpec((1, OUT_DIM), lambda i: (0, 0)),
            pl.BlockSpec((1, 1, RB), lambda i: (i, 0, 0)),
        ],
        out_specs=pl.BlockSpec((GRAPHS, OUT_DIM), lambda i: (0, 0)),
        out_shape=jax.ShapeDtypeStruct((GRAPHS, OUT_DIM), jnp.float32),
        scratch_shapes=[
            pltpu.VMEM((GRAPHS, OUT_DIM), jnp.float32),
            pltpu.VMEM((GRAPHS, OUT_DIM), jnp.float32),
        ],
    )(q, y1, degp, b2, batch3)


def kernel(x, edge_index, batch, W1, b1, W2, b2):
    src = edge_index[0]
    dst = edge_index[1]
    e = src.shape[0]
    src_f = jnp.concatenate([src, jnp.zeros((EPAD - e,), jnp.int32)])
    dst_f = jnp.concatenate([dst, jnp.full((EPAD - e,), NODES, jnp.int32)])
    dst_p = dst_f.reshape(NW, PH, CHP, CHUNK)
    srcA = src_f[:E0].reshape(NS, AP0, ACH, CHUNK)
    dstA = dst_f[:E0].reshape(NS, AP0, ACH, CHUNK)
    ones128 = jnp.ones((CHUNK, IN_DIM), jnp.float32)
    zerosf = jnp.zeros((NPAD, IN_DIM), jnp.float32)

    degp = _deg_call(dst_p, ones128, zerosf)
    y0 = _scale_call(x, degp)
    p = _agg_call(y0, srcA, dstA, zerosf)
    y1 = _mid_call(p, y0, degp, W1, b1.reshape(1, HID_DIM), W2)
    q = _agg_call(y1, srcA, dstA, zerosf)
    return _final_call(q, y1, degp, b2.reshape(1, OUT_DIM), batch.reshape(GRID, 1, RB))


# speedup vs baseline: 3.0067x; 3.0067x over previous
import functools

import jax
import jax.numpy as jnp
from jax import lax
from jax.experimental import pallas as pl
from jax.experimental.pallas import tpu as pltpu
from jax.experimental.pallas import tpu_sc as plsc

NODES = 10000
IN_DIM = 128
HID_DIM = 256
OUT_DIM = 128
GRAPHS = 16

NC = 2
NS = 16
NW = NC * NS
CHUNK = 128
CH = 80
PH = 2
CHP = CH // PH
EPAD = NW * CH * CHUNK
NPAD = NODES + 112
RPT = NPAD // NS
RB = 1000
GRID = NODES // RB


def _sc_mesh():
    return plsc.VectorSubcoreMesh(core_axis_name="c", subcore_axis_name="s")


def _deg_call(dst_p, ones128, zerosf):
    @functools.partial(
        pl.kernel,
        out_type=jax.ShapeDtypeStruct((NC, NPAD, IN_DIM), jnp.float32),
        mesh=_sc_mesh(),
        scratch_types=[
            pltpu.VMEM((PH, CHP, CHUNK), jnp.int32),
            pltpu.VMEM((CHUNK, IN_DIM), jnp.float32),
            pltpu.VMEM_SHARED((NPAD, IN_DIM), jnp.float32),
        ],
    )
    def k(dst_hbm, ones_hbm, zeros_hbm, out_hbm, dst_v, ones_v, acc):
        cid = lax.axis_index("c")
        sid = lax.axis_index("s")
        wid = cid * NS + sid

        @pl.when(sid == 0)
        def _():
            pltpu.sync_copy(zeros_hbm, acc)

        pltpu.sync_copy(dst_hbm.at[wid], dst_v)
        pltpu.sync_copy(ones_hbm, ones_v)
        plsc.subcore_barrier()

        def body(p, c):
            def inner(j, c2):
                pltpu.sync_copy(ones_v, acc.at[dst_v.at[p, j]], add=True)
                return c2

            return lax.fori_loop(0, CHP, inner, c)

        lax.fori_loop(0, PH, body, 0)
        plsc.subcore_barrier()
        pltpu.sync_copy(
            acc.at[pl.ds(sid * RPT, RPT)],
            out_hbm.at[cid, pl.ds(sid * RPT, RPT)],
        )

    return k(dst_p, ones128, zerosf)


def _agg_call(y, src_p, dst_p, zerosf):
    @functools.partial(
        pl.kernel,
        out_type=jax.ShapeDtypeStruct((NC, NPAD, IN_DIM), jnp.float32),
        mesh=_sc_mesh(),
        scratch_types=[
            pltpu.VMEM((CHP, CHUNK), jnp.int32),
            pltpu.VMEM((CHP, CHUNK), jnp.int32),
            pltpu.VMEM((CHUNK, IN_DIM), jnp.float32),
            pltpu.VMEM((CHUNK, IN_DIM), jnp.float32),
            pltpu.SemaphoreType.DMA,
            pltpu.SemaphoreType.DMA,
            pltpu.VMEM_SHARED((NPAD, IN_DIM), jnp.float32),
        ],
    )
    def k(y_hbm, src_hbm, dst_hbm, zeros_hbm, out_hbm,
          src_v, dst_v, rows0, rows1, sem0, sem1, acc):
        cid = lax.axis_index("c")
        sid = lax.axis_index("s")
        wid = cid * NS + sid

        @pl.when(sid == 0)
        def _():
            pltpu.sync_copy(zeros_hbm, acc)

        plsc.subcore_barrier()

        def phase(p, c):
            pltpu.sync_copy(src_hbm.at[wid, p], src_v)
            pltpu.sync_copy(dst_hbm.at[wid, p], dst_v)
            pltpu.async_copy(y_hbm.at[src_v.at[0]], rows0, sem0)

            def body(t, c2):
                j0 = 2 * t
                j1 = j0 + 1
                pltpu.async_copy(y_hbm.at[src_v.at[j1]], rows1, sem1)
                pltpu.make_async_copy(y_hbm.at[src_v.at[j0]], rows0, sem0).wait()
                pltpu.sync_copy(rows0, acc.at[dst_v.at[j0]], add=True)

                @pl.when(j0 + 2 < CHP)
                def _():
                    pltpu.async_copy(y_hbm.at[src_v.at[j0 + 2]], rows0, sem0)

                pltpu.make_async_copy(y_hbm.at[src_v.at[j1]], rows1, sem1).wait()
                pltpu.sync_copy(rows1, acc.at[dst_v.at[j1]], add=True)
                return c2

            return lax.fori_loop(0, CHP // 2, body, c)

        lax.fori_loop(0, PH, phase, 0)
        plsc.subcore_barrier()
        pltpu.sync_copy(
            acc.at[pl.ds(sid * RPT, RPT)],
            out_hbm.at[cid, pl.ds(sid * RPT, RPT)],
        )

    return k(y, src_p, dst_p, zerosf)


def _dinv_block(degp_block):
    deg = degp_block[0, :, :1] + degp_block[1, :, :1] + 1.0
    return lax.rsqrt(deg)


def _scale_call(x, degp):
    def body(x_ref, degp_ref, y_ref):
        dinv = _dinv_block(degp_ref[...])
        y_ref[...] = x_ref[...] * dinv

    return pl.pallas_call(
        body,
        grid=(GRID,),
        in_specs=[
            pl.BlockSpec((RB, IN_DIM), lambda i: (i, 0)),
            pl.BlockSpec((NC, RB, IN_DIM), lambda i: (0, i, 0)),
        ],
        out_specs=pl.BlockSpec((RB, IN_DIM), lambda i: (i, 0)),
        out_shape=jax.ShapeDtypeStruct((NODES, IN_DIM), jnp.float32),
    )(x, degp)


def _mid_call(p, y0, degp, W1, b1, W2):
    def body(p_ref, y0_ref, degp_ref, w1_ref, b1_ref, w2_ref, y1_ref):
        dinv = _dinv_block(degp_ref[...])
        pv = p_ref[...]
        t0 = dinv * (pv[0] + pv[1] + y0_ref[...])
        h = jnp.maximum(
            jnp.dot(t0, w1_ref[...], preferred_element_type=jnp.float32)
            + b1_ref[...],
            0.0,
        )
        z = jnp.dot(h, w2_ref[...], preferred_element_type=jnp.float32)
        y1_ref[...] = dinv * z

    return pl.pallas_call(
        body,
        grid=(GRID,),
        in_specs=[
            pl.BlockSpec((NC, RB, IN_DIM), lambda i: (0, i, 0)),
            pl.BlockSpec((RB, IN_DIM), lambda i: (i, 0)),
            pl.BlockSpec((NC, RB, IN_DIM), lambda i: (0, i, 0)),
            pl.BlockSpec((IN_DIM, HID_DIM), lambda i: (0, 0)),
            pl.BlockSpec((1, HID_DIM), lambda i: (0, 0)),
            pl.BlockSpec((HID_DIM, OUT_DIM), lambda i: (0, 0)),
        ],
        out_specs=pl.BlockSpec((RB, OUT_DIM), lambda i: (i, 0)),
        out_shape=jax.ShapeDtypeStruct((NODES, OUT_DIM), jnp.float32),
    )(p, y0, degp, W1, b1, W2)


def _final_call(q, y1, degp, b2, batch3):
    def body(q_ref, y1_ref, degp_ref, b2_ref, batch_ref, out_ref, sums, cnts):
        i = pl.program_id(0)

        @pl.when(i == 0)
        def _():
            sums[...] = jnp.zeros_like(sums)
            cnts[...] = jnp.zeros_like(cnts)

        dinv = _dinv_block(degp_ref[...])
        qv = q_ref[...]
        t1 = dinv * (qv[0] + qv[1] + y1_ref[...])
        h2 = jnp.maximum(t1 + b2_ref[...], 0.0)
        b = batch_ref[...].reshape(1, RB)
        iota = lax.broadcasted_iota(jnp.int32, (GRAPHS, RB), 0)
        onehot = (jnp.broadcast_to(b, (GRAPHS, RB)) == iota).astype(jnp.float32)
        sums[...] += jnp.dot(onehot, h2, preferred_element_type=jnp.float32)
        cnts[...] += jnp.broadcast_to(
            jnp.sum(onehot, axis=1, keepdims=True), (GRAPHS, OUT_DIM)
        )

        @pl.when(i == GRID - 1)
        def _():
            out_ref[...] = sums[...] / jnp.maximum(cnts[...], 1.0)

    return pl.pallas_call(
        body,
        grid=(GRID,),
        in_specs=[
            pl.BlockSpec((NC, RB, IN_DIM), lambda i: (0, i, 0)),
            pl.BlockSpec((RB, OUT_DIM), lambda i: (i, 0)),
            pl.BlockSpec((NC, RB, IN_DIM), lambda i: (0, i, 0)),
            pl.BlockSpec((1, OUT_DIM), lambda i: (0, 0)),
            pl.BlockSpec((1, 1, RB), lambda i: (i, 0, 0)),
        ],
        out_specs=pl.BlockSpec((GRAPHS, OUT_DIM), lambda i: (0, 0)),
        out_shape=jax.ShapeDtypeStruct((GRAPHS, OUT_DIM), jnp.float32),
        scratch_shapes=[
            pltpu.VMEM((GRAPHS, OUT_DIM), jnp.float32),
            pltpu.VMEM((GRAPHS, OUT_DIM), jnp.float32),
        ],
    )(q, y1, degp, b2, batch3)


def kernel(x, edge_index, batch, W1, b1, W2, b2):
    src = edge_index[0]
    dst = edge_index[1]
    e = src.shape[0]
    pad_src = jnp.arange(EPAD - e, dtype=jnp.int32) % NODES
    src_f = jnp.concatenate([src, pad_src])
    dst_f = jnp.concatenate([dst, jnp.full((EPAD - e,), NODES, jnp.int32)])
    src_p = src_f.reshape(NW, PH, CHP, CHUNK)
    dst_p = dst_f.reshape(NW, PH, CHP, CHUNK)
    ones128 = jnp.ones((CHUNK, IN_DIM), jnp.float32)
    zerosf = jnp.zeros((NPAD, IN_DIM), jnp.float32)

    degp = _deg_call(dst_p, ones128, zerosf)
    y0 = _scale_call(x, degp)
    p = _agg_call(y0, src_p, dst_p, zerosf)
    y1 = _mid_call(p, y0, degp, W1, b1.reshape(1, HID_DIM), W2)
    q = _agg_call(y1, src_p, dst_p, zerosf)
    return _final_call(q, y1, degp, b2.reshape(1, OUT_DIM), batch.reshape(GRID, 1, RB))


# speedup vs baseline: 3.0149x; 1.0027x over previous
import functools

import jax
import jax.numpy as jnp
from jax import lax
from jax.experimental import pallas as pl
from jax.experimental.pallas import tpu as pltpu
from jax.experimental.pallas import tpu_sc as plsc

NODES = 10000
IN_DIM = 128
HID_DIM = 256
OUT_DIM = 128
GRAPHS = 16

NC = 2
NS = 16
NW = NC * NS
CHUNK = 128
CH = 80
PH = 2
CHP = CH // PH
ACK = 64
NB = 4
APH = 4
ACH = (CH * CHUNK) // (APH * ACK)
EPAD = NW * CH * CHUNK
NPAD = NODES + 112
RPT = NPAD // NS
RB = 1000
GRID = NODES // RB


def _sc_mesh():
    return plsc.VectorSubcoreMesh(core_axis_name="c", subcore_axis_name="s")


def _deg_call(dst_p, ones128, zerosf):
    @functools.partial(
        pl.kernel,
        out_type=jax.ShapeDtypeStruct((NC, NPAD, IN_DIM), jnp.float32),
        mesh=_sc_mesh(),
        scratch_types=[
            pltpu.VMEM((PH, CHP, CHUNK), jnp.int32),
            pltpu.VMEM((CHUNK, IN_DIM), jnp.float32),
            pltpu.VMEM_SHARED((NPAD, IN_DIM), jnp.float32),
        ],
    )
    def k(dst_hbm, ones_hbm, zeros_hbm, out_hbm, dst_v, ones_v, acc):
        cid = lax.axis_index("c")
        sid = lax.axis_index("s")
        wid = cid * NS + sid

        @pl.when(sid == 0)
        def _():
            pltpu.sync_copy(zeros_hbm, acc)

        pltpu.sync_copy(dst_hbm.at[wid], dst_v)
        pltpu.sync_copy(ones_hbm, ones_v)
        plsc.subcore_barrier()

        def body(p, c):
            def inner(j, c2):
                pltpu.sync_copy(ones_v, acc.at[dst_v.at[p, j]], add=True)
                return c2

            return lax.fori_loop(0, CHP, inner, c)

        lax.fori_loop(0, PH, body, 0)
        plsc.subcore_barrier()
        pltpu.sync_copy(
            acc.at[pl.ds(sid * RPT, RPT)],
            out_hbm.at[cid, pl.ds(sid * RPT, RPT)],
        )

    return k(dst_p, ones128, zerosf)


def _agg_call(y, src_p, dst_p, zerosf):
    @functools.partial(
        pl.kernel,
        out_type=jax.ShapeDtypeStruct((NC, NPAD, IN_DIM), jnp.float32),
        mesh=_sc_mesh(),
        scratch_types=[
            pltpu.VMEM((ACH, ACK), jnp.int32),
            pltpu.VMEM((ACH, ACK), jnp.int32),
            [pltpu.VMEM((ACK, IN_DIM), jnp.float32)] * NB,
            [pltpu.SemaphoreType.DMA] * NB,
            pltpu.VMEM_SHARED((NPAD, IN_DIM), jnp.float32),
        ],
    )
    def k(y_hbm, src_hbm, dst_hbm, zeros_hbm, out_hbm,
          src_v, dst_v, rows, sems, acc):
        cid = lax.axis_index("c")
        sid = lax.axis_index("s")
        wid = cid * NS + sid

        @pl.when(sid == 0)
        def _():
            pltpu.sync_copy(zeros_hbm, acc)

        plsc.subcore_barrier()

        def phase(p, c):
            pltpu.sync_copy(src_hbm.at[wid, p], src_v)
            pltpu.sync_copy(dst_hbm.at[wid, p], dst_v)
            for b in range(NB - 1):
                pltpu.async_copy(y_hbm.at[src_v.at[b]], rows[b], sems[b])

            def body(t, c2):
                c3 = c2
                for b in range(NB):
                    j = NB * t + b
                    pltpu.make_async_copy(y_hbm.at[src_v.at[j]], rows[b], sems[b]).wait()
                    pltpu.sync_copy(rows[b], acc.at[dst_v.at[j]], add=True)
                    bn = (b + NB - 1) % NB

                    @pl.when(j + NB - 1 < ACH)
                    def _():
                        pltpu.async_copy(
                            y_hbm.at[src_v.at[j + NB - 1]], rows[bn], sems[bn])
                return c3

            return lax.fori_loop(0, ACH // NB, body, c)

        lax.fori_loop(0, APH, phase, 0)
        plsc.subcore_barrier()
        pltpu.sync_copy(
            acc.at[pl.ds(sid * RPT, RPT)],
            out_hbm.at[cid, pl.ds(sid * RPT, RPT)],
        )

    return k(y, src_p, dst_p, zerosf)


def _dinv_block(degp_block):
    deg = degp_block[0, :, :1] + degp_block[1, :, :1] + 1.0
    return lax.rsqrt(deg)


def _scale_call(x, degp):
    def body(x_ref, degp_ref, y_ref):
        dinv = _dinv_block(degp_ref[...])
        y_ref[...] = x_ref[...] * dinv

    return pl.pallas_call(
        body,
        grid=(GRID,),
        in_specs=[
            pl.BlockSpec((RB, IN_DIM), lambda i: (i, 0)),
            pl.BlockSpec((NC, RB, IN_DIM), lambda i: (0, i, 0)),
        ],
        out_specs=pl.BlockSpec((RB, IN_DIM), lambda i: (i, 0)),
        out_shape=jax.ShapeDtypeStruct((NODES, IN_DIM), jnp.float32),
    )(x, degp)


def _mid_call(p, y0, degp, W1, b1, W2):
    def body(p_ref, y0_ref, degp_ref, w1_ref, b1_ref, w2_ref, y1_ref):
        dinv = _dinv_block(degp_ref[...])
        pv = p_ref[...]
        t0 = dinv * (pv[0] + pv[1] + y0_ref[...])
        h = jnp.maximum(
            jnp.dot(t0, w1_ref[...], preferred_element_type=jnp.float32)
            + b1_ref[...],
            0.0,
        )
        z = jnp.dot(h, w2_ref[...], preferred_element_type=jnp.float32)
        y1_ref[...] = dinv * z

    return pl.pallas_call(
        body,
        grid=(GRID,),
        in_specs=[
            pl.BlockSpec((NC, RB, IN_DIM), lambda i: (0, i, 0)),
            pl.BlockSpec((RB, IN_DIM), lambda i: (i, 0)),
            pl.BlockSpec((NC, RB, IN_DIM), lambda i: (0, i, 0)),
            pl.BlockSpec((IN_DIM, HID_DIM), lambda i: (0, 0)),
            pl.BlockSpec((1, HID_DIM), lambda i: (0, 0)),
            pl.BlockSpec((HID_DIM, OUT_DIM), lambda i: (0, 0)),
        ],
        out_specs=pl.BlockSpec((RB, OUT_DIM), lambda i: (i, 0)),
        out_shape=jax.ShapeDtypeStruct((NODES, OUT_DIM), jnp.float32),
    )(p, y0, degp, W1, b1, W2)


def _final_call(q, y1, degp, b2, batch3):
    def body(q_ref, y1_ref, degp_ref, b2_ref, batch_ref, out_ref, sums, cnts):
        i = pl.program_id(0)

        @pl.when(i == 0)
        def _():
            sums[...] = jnp.zeros_like(sums)
            cnts[...] = jnp.zeros_like(cnts)

        dinv = _dinv_block(degp_ref[...])
        qv = q_ref[...]
        t1 = dinv * (qv[0] + qv[1] + y1_ref[...])
        h2 = jnp.maximum(t1 + b2_ref[...], 0.0)
        b = batch_ref[...].reshape(1, RB)
        iota = lax.broadcasted_iota(jnp.int32, (GRAPHS, RB), 0)
        onehot = (jnp.broadcast_to(b, (GRAPHS, RB)) == iota).astype(jnp.float32)
        sums[...] += jnp.dot(onehot, h2, preferred_element_type=jnp.float32)
        cnts[...] += jnp.broadcast_to(
            jnp.sum(onehot, axis=1, keepdims=True), (GRAPHS, OUT_DIM)
        )

        @pl.when(i == GRID - 1)
        def _():
            out_ref[...] = sums[...] / jnp.maximum(cnts[...], 1.0)

    return pl.pallas_call(
        body,
        grid=(GRID,),
        in_specs=[
            pl.BlockSpec((NC, RB, IN_DIM), lambda i: (0, i, 0)),
            pl.BlockSpec((RB, OUT_DIM), lambda i: (i, 0)),
            pl.BlockSpec((NC, RB, IN_DIM), lambda i: (0, i, 0)),
            pl.BlockSpec((1, OUT_DIM), lambda i: (0, 0)),
            pl.BlockSpec((1, 1, RB), lambda i: (i, 0, 0)),
        ],
        out_specs=pl.BlockSpec((GRAPHS, OUT_DIM), lambda i: (0, 0)),
        out_shape=jax.ShapeDtypeStruct((GRAPHS, OUT_DIM), jnp.float32),
        scratch_shapes=[
            pltpu.VMEM((GRAPHS, OUT_DIM), jnp.float32),
            pltpu.VMEM((GRAPHS, OUT_DIM), jnp.float32),
        ],
    )(q, y1, degp, b2, batch3)


def kernel(x, edge_index, batch, W1, b1, W2, b2):
    src = edge_index[0]
    dst = edge_index[1]
    e = src.shape[0]
    pad_src = jnp.arange(EPAD - e, dtype=jnp.int32) % NODES
    src_f = jnp.concatenate([src, pad_src])
    dst_f = jnp.concatenate([dst, jnp.full((EPAD - e,), NODES, jnp.int32)])
    src_p = src_f.reshape(NW, PH, CHP, CHUNK)
    dst_p = dst_f.reshape(NW, PH, CHP, CHUNK)
    src_a = src_f.reshape(NW, APH, ACH, ACK)
    dst_a = dst_f.reshape(NW, APH, ACH, ACK)
    ones128 = jnp.ones((CHUNK, IN_DIM), jnp.float32)
    zerosf = jnp.zeros((NPAD, IN_DIM), jnp.float32)

    degp = _deg_call(dst_p, ones128, zerosf)
    y0 = _scale_call(x, degp)
    p = _agg_call(y0, src_a, dst_a, zerosf)
    y1 = _mid_call(p, y0, degp, W1, b1.reshape(1, HID_DIM), W2)
    q = _agg_call(y1, src_a, dst_a, zerosf)
    return _final_call(q, y1, degp, b2.reshape(1, OUT_DIM), batch.reshape(GRID, 1, RB))


# speedup vs baseline: 3.0595x; 1.0148x over previous
import functools

import jax
import jax.numpy as jnp
from jax import lax
from jax.experimental import pallas as pl
from jax.experimental.pallas import tpu as pltpu
from jax.experimental.pallas import tpu_sc as plsc

NODES = 10000
IN_DIM = 128
HID_DIM = 256
OUT_DIM = 128
GRAPHS = 16

NC = 2
NS = 16
NW = NC * NS
CHUNK = 128
CH = 80
PH = 2
CHP = CH // PH
ACK = 64
NB = 4
APH = 4
ACH = (CH * CHUNK) // (APH * ACK)
EPAD = NW * CH * CHUNK
NPAD = NODES + 112
RPT = NPAD // NS
RB = 2000
GRID = NODES // RB


def _sc_mesh():
    return plsc.VectorSubcoreMesh(core_axis_name="c", subcore_axis_name="s")


def _deg_call(dst_p, ones128, zerosf):
    @functools.partial(
        pl.kernel,
        out_type=jax.ShapeDtypeStruct((NC, NPAD, IN_DIM), jnp.float32),
        mesh=_sc_mesh(),
        scratch_types=[
            pltpu.VMEM((APH, ACH, ACK), jnp.int32),
            pltpu.VMEM((ACK, IN_DIM), jnp.float32),
            pltpu.VMEM_SHARED((NPAD, IN_DIM), jnp.float32),
        ],
    )
    def k(dst_hbm, ones_hbm, zeros_hbm, out_hbm, dst_v, ones_v, acc):
        cid = lax.axis_index("c")
        sid = lax.axis_index("s")
        wid = cid * NS + sid

        @pl.when(sid == 0)
        def _():
            pltpu.sync_copy(zeros_hbm, acc)

        pltpu.sync_copy(dst_hbm.at[wid], dst_v)
        pltpu.sync_copy(ones_hbm, ones_v)
        plsc.subcore_barrier()

        def body(p, c):
            def inner(j, c2):
                pltpu.sync_copy(ones_v, acc.at[dst_v.at[p, j]], add=True)
                return c2

            return lax.fori_loop(0, ACH, inner, c)

        lax.fori_loop(0, APH, body, 0)
        plsc.subcore_barrier()
        pltpu.sync_copy(
            acc.at[pl.ds(sid * RPT, RPT)],
            out_hbm.at[cid, pl.ds(sid * RPT, RPT)],
        )

    return k(dst_p, ones128, zerosf)


def _agg_call(y, src_p, dst_p, zerosf):
    @functools.partial(
        pl.kernel,
        out_type=jax.ShapeDtypeStruct((NC, NPAD, IN_DIM), jnp.float32),
        mesh=_sc_mesh(),
        scratch_types=[
            pltpu.VMEM((ACH, ACK), jnp.int32),
            pltpu.VMEM((ACH, ACK), jnp.int32),
            [pltpu.VMEM((ACK, IN_DIM), jnp.float32)] * NB,
            [pltpu.SemaphoreType.DMA] * NB,
            pltpu.VMEM_SHARED((NPAD, IN_DIM), jnp.float32),
        ],
    )
    def k(y_hbm, src_hbm, dst_hbm, zeros_hbm, out_hbm,
          src_v, dst_v, rows, sems, acc):
        cid = lax.axis_index("c")
        sid = lax.axis_index("s")
        wid = cid * NS + sid

        @pl.when(sid == 0)
        def _():
            pltpu.sync_copy(zeros_hbm, acc)

        plsc.subcore_barrier()

        def phase(p, c):
            pltpu.sync_copy(src_hbm.at[wid, p], src_v)
            pltpu.sync_copy(dst_hbm.at[wid, p], dst_v)
            for b in range(NB - 1):
                pltpu.async_copy(y_hbm.at[src_v.at[b]], rows[b], sems[b])

            def body(t, c2):
                c3 = c2
                for b in range(NB):
                    j = NB * t + b
                    pltpu.make_async_copy(y_hbm.at[src_v.at[j]], rows[b], sems[b]).wait()
                    pltpu.sync_copy(rows[b], acc.at[dst_v.at[j]], add=True)
                    bn = (b + NB - 1) % NB

                    @pl.when(j + NB - 1 < ACH)
                    def _():
                        pltpu.async_copy(
                            y_hbm.at[src_v.at[j + NB - 1]], rows[bn], sems[bn])
                return c3

            return lax.fori_loop(0, ACH // NB, body, c)

        lax.fori_loop(0, APH, phase, 0)
        plsc.subcore_barrier()
        pltpu.sync_copy(
            acc.at[pl.ds(sid * RPT, RPT)],
            out_hbm.at[cid, pl.ds(sid * RPT, RPT)],
        )

    return k(y, src_p, dst_p, zerosf)


def _dinv_block(degp_block):
    deg = degp_block[0, :, :1] + degp_block[1, :, :1] + 1.0
    return lax.rsqrt(deg)


def _scale_call(x, degp):
    def body(x_ref, degp_ref, y_ref):
        dinv = _dinv_block(degp_ref[...])
        y_ref[...] = x_ref[...] * dinv

    return pl.pallas_call(
        body,
        grid=(GRID,),
        in_specs=[
            pl.BlockSpec((RB, IN_DIM), lambda i: (i, 0)),
            pl.BlockSpec((NC, RB, IN_DIM), lambda i: (0, i, 0)),
        ],
        out_specs=pl.BlockSpec((RB, IN_DIM), lambda i: (i, 0)),
        out_shape=jax.ShapeDtypeStruct((NODES, IN_DIM), jnp.float32),
    )(x, degp)


def _mid_call(p, y0, degp, W1, b1, W2):
    def body(p_ref, y0_ref, degp_ref, w1_ref, b1_ref, w2_ref, y1_ref):
        dinv = _dinv_block(degp_ref[...])
        pv = p_ref[...]
        t0 = dinv * (pv[0] + pv[1] + y0_ref[...])
        h = jnp.maximum(
            jnp.dot(t0, w1_ref[...], preferred_element_type=jnp.float32)
            + b1_ref[...],
            0.0,
        )
        z = jnp.dot(h, w2_ref[...], preferred_element_type=jnp.float32)
        y1_ref[...] = dinv * z

    return pl.pallas_call(
        body,
        grid=(GRID,),
        in_specs=[
            pl.BlockSpec((NC, RB, IN_DIM), lambda i: (0, i, 0)),
            pl.BlockSpec((RB, IN_DIM), lambda i: (i, 0)),
            pl.BlockSpec((NC, RB, IN_DIM), lambda i: (0, i, 0)),
            pl.BlockSpec((IN_DIM, HID_DIM), lambda i: (0, 0)),
            pl.BlockSpec((1, HID_DIM), lambda i: (0, 0)),
            pl.BlockSpec((HID_DIM, OUT_DIM), lambda i: (0, 0)),
        ],
        out_specs=pl.BlockSpec((RB, OUT_DIM), lambda i: (i, 0)),
        out_shape=jax.ShapeDtypeStruct((NODES, OUT_DIM), jnp.float32),
    )(p, y0, degp, W1, b1, W2)


def _final_call(q, y1, degp, b2, batch3):
    def body(q_ref, y1_ref, degp_ref, b2_ref, batch_ref, out_ref, sums, cnts):
        i = pl.program_id(0)

        @pl.when(i == 0)
        def _():
            sums[...] = jnp.zeros_like(sums)
            cnts[...] = jnp.zeros_like(cnts)

        dinv = _dinv_block(degp_ref[...])
        qv = q_ref[...]
        t1 = dinv * (qv[0] + qv[1] + y1_ref[...])
        h2 = jnp.maximum(t1 + b2_ref[...], 0.0)
        b = batch_ref[...].reshape(1, RB)
        iota = lax.broadcasted_iota(jnp.int32, (GRAPHS, RB), 0)
        onehot = (jnp.broadcast_to(b, (GRAPHS, RB)) == iota).astype(jnp.float32)
        sums[...] += jnp.dot(onehot, h2, preferred_element_type=jnp.float32)
        cnts[...] += jnp.broadcast_to(
            jnp.sum(onehot, axis=1, keepdims=True), (GRAPHS, OUT_DIM)
        )

        @pl.when(i == GRID - 1)
        def _():
            out_ref[...] = sums[...] / jnp.maximum(cnts[...], 1.0)

    return pl.pallas_call(
        body,
        grid=(GRID,),
        in_specs=[
            pl.BlockSpec((NC, RB, IN_DIM), lambda i: (0, i, 0)),
            pl.BlockSpec((RB, OUT_DIM), lambda i: (i, 0)),
            pl.BlockSpec((NC, RB, IN_DIM), lambda i: (0, i, 0)),
            pl.BlockSpec((1, OUT_DIM), lambda i: (0, 0)),
            pl.BlockSpec((1, 1, RB), lambda i: (i, 0, 0)),
        ],
        out_specs=pl.BlockSpec((GRAPHS, OUT_DIM), lambda i: (0, 0)),
        out_shape=jax.ShapeDtypeStruct((GRAPHS, OUT_DIM), jnp.float32),
        scratch_shapes=[
            pltpu.VMEM((GRAPHS, OUT_DIM), jnp.float32),
            pltpu.VMEM((GRAPHS, OUT_DIM), jnp.float32),
        ],
    )(q, y1, degp, b2, batch3)


def kernel(x, edge_index, batch, W1, b1, W2, b2):
    src = edge_index[0]
    dst = edge_index[1]
    e = src.shape[0]
    pad_src = jnp.arange(EPAD - e, dtype=jnp.int32)
    src_f = jnp.concatenate([src, pad_src])
    dst_f = jnp.concatenate([dst, jnp.full((EPAD - e,), NODES, jnp.int32)])
    src_a = src_f.reshape(NW, APH, ACH, ACK)
    dst_a = dst_f.reshape(NW, APH, ACH, ACK)
    ones128 = jnp.ones((ACK, IN_DIM), jnp.float32)
    zerosf = jnp.zeros((NPAD, IN_DIM), jnp.float32)

    degp = _deg_call(dst_a, ones128, zerosf)
    y0 = _scale_call(x, degp)
    p = _agg_call(y0, src_a, dst_a, zerosf)
    y1 = _mid_call(p, y0, degp, W1, b1.reshape(1, HID_DIM), W2)
    q = _agg_call(y1, src_a, dst_a, zerosf)
    return _final_call(q, y1, degp, b2.reshape(1, OUT_DIM), batch.reshape(GRID, 1, RB))


# speedup vs baseline: 3.0970x; 1.0123x over previous
import functools

import jax
import jax.numpy as jnp
from jax import lax
from jax.experimental import pallas as pl
from jax.experimental.pallas import tpu as pltpu
from jax.experimental.pallas import tpu_sc as plsc

NODES = 10000
IN_DIM = 128
HID_DIM = 256
OUT_DIM = 128
GRAPHS = 16

NC = 2
NS = 16
NW = NC * NS
CHUNK = 128
CH = 80
PH = 2
CHP = CH // PH
ACK = 64
NB = 4
APH = 4
ACH = (CH * CHUNK) // (APH * ACK)
EPAD = NW * CH * CHUNK
NPAD = NODES + 112
RPT = NPAD // NS
RB = 2000
GRID = NODES // RB


def _sc_mesh():
    return plsc.VectorSubcoreMesh(core_axis_name="c", subcore_axis_name="s")


def _deg_call(edges, ones128, zerosf):
    @functools.partial(
        pl.kernel,
        out_type=jax.ShapeDtypeStruct((NC, NPAD, IN_DIM), jnp.float32),
        mesh=_sc_mesh(),
        scratch_types=[
            pltpu.VMEM((ACH, ACK), jnp.int32),
            pltpu.VMEM((ACK, IN_DIM), jnp.float32),
            pltpu.VMEM_SHARED((NPAD, IN_DIM), jnp.float32),
        ],
    )
    def k(edges_hbm, ones_hbm, zeros_hbm, out_hbm, dst_v, ones_v, acc):
        cid = lax.axis_index("c")
        sid = lax.axis_index("s")
        wid = cid * NS + sid

        @pl.when(sid == 0)
        def _():
            pltpu.sync_copy(zeros_hbm, acc)

        pltpu.sync_copy(ones_hbm, ones_v)
        plsc.subcore_barrier()

        def body(p, c):
            pltpu.sync_copy(
                edges_hbm.at[1, pl.ds((wid * APH + p) * ACH, ACH)], dst_v)

            def inner(j, c2):
                pltpu.sync_copy(ones_v, acc.at[dst_v.at[j]], add=True)
                return c2

            return lax.fori_loop(0, ACH, inner, c)

        lax.fori_loop(0, APH, body, 0)
        plsc.subcore_barrier()
        pltpu.sync_copy(
            acc.at[pl.ds(sid * RPT, RPT)],
            out_hbm.at[cid, pl.ds(sid * RPT, RPT)],
        )

    return k(edges, ones128, zerosf)


def _agg_call(y, edges, zerosf):
    @functools.partial(
        pl.kernel,
        out_type=jax.ShapeDtypeStruct((NC, NPAD, IN_DIM), jnp.float32),
        mesh=_sc_mesh(),
        scratch_types=[
            pltpu.VMEM((ACH, ACK), jnp.int32),
            pltpu.VMEM((ACH, ACK), jnp.int32),
            [pltpu.VMEM((ACK, IN_DIM), jnp.float32)] * NB,
            [pltpu.SemaphoreType.DMA] * NB,
            pltpu.VMEM_SHARED((NPAD, IN_DIM), jnp.float32),
        ],
    )
    def k(y_hbm, edges_hbm, zeros_hbm, out_hbm,
          src_v, dst_v, rows, sems, acc):
        cid = lax.axis_index("c")
        sid = lax.axis_index("s")
        wid = cid * NS + sid

        @pl.when(sid == 0)
        def _():
            pltpu.sync_copy(zeros_hbm, acc)

        plsc.subcore_barrier()

        def phase(p, c):
            base = (wid * APH + p) * ACH
            pltpu.sync_copy(edges_hbm.at[0, pl.ds(base, ACH)], src_v)
            pltpu.sync_copy(edges_hbm.at[1, pl.ds(base, ACH)], dst_v)
            for b in range(NB - 1):
                pltpu.async_copy(y_hbm.at[src_v.at[b]], rows[b], sems[b])

            def body(t, c2):
                c3 = c2
                for b in range(NB):
                    j = NB * t + b
                    pltpu.make_async_copy(y_hbm.at[src_v.at[j]], rows[b], sems[b]).wait()
                    pltpu.sync_copy(rows[b], acc.at[dst_v.at[j]], add=True)
                    bn = (b + NB - 1) % NB

                    @pl.when(j + NB - 1 < ACH)
                    def _():
                        pltpu.async_copy(
                            y_hbm.at[src_v.at[j + NB - 1]], rows[bn], sems[bn])
                return c3

            return lax.fori_loop(0, ACH // NB, body, c)

        lax.fori_loop(0, APH, phase, 0)
        plsc.subcore_barrier()
        pltpu.sync_copy(
            acc.at[pl.ds(sid * RPT, RPT)],
            out_hbm.at[cid, pl.ds(sid * RPT, RPT)],
        )

    return k(y, edges, zerosf)


def _dinv_block(degp_block):
    deg = degp_block[0, :, :1] + degp_block[1, :, :1] + 1.0
    return lax.rsqrt(deg)


def _scale_call(x, degp):
    def body(x_ref, degp_ref, y_ref):
        dinv = _dinv_block(degp_ref[...])
        y_ref[...] = x_ref[...] * dinv

    return pl.pallas_call(
        body,
        grid=(GRID,),
        in_specs=[
            pl.BlockSpec((RB, IN_DIM), lambda i: (i, 0)),
            pl.BlockSpec((NC, RB, IN_DIM), lambda i: (0, i, 0)),
        ],
        out_specs=pl.BlockSpec((RB, IN_DIM), lambda i: (i, 0)),
        out_shape=jax.ShapeDtypeStruct((NODES, IN_DIM), jnp.float32),
    )(x, degp)


def _mid_call(p, y0, degp, W1, b1, W2):
    def body(p_ref, y0_ref, degp_ref, w1_ref, b1_ref, w2_ref, y1_ref):
        dinv = _dinv_block(degp_ref[...])
        pv = p_ref[...]
        t0 = dinv * (pv[0] + pv[1] + y0_ref[...])
        h = jnp.maximum(
            jnp.dot(t0, w1_ref[...], preferred_element_type=jnp.float32)
            + b1_ref[...],
            0.0,
        )
        z = jnp.dot(h, w2_ref[...], preferred_element_type=jnp.float32)
        y1_ref[...] = dinv * z

    return pl.pallas_call(
        body,
        grid=(GRID,),
        in_specs=[
            pl.BlockSpec((NC, RB, IN_DIM), lambda i: (0, i, 0)),
            pl.BlockSpec((RB, IN_DIM), lambda i: (i, 0)),
            pl.BlockSpec((NC, RB, IN_DIM), lambda i: (0, i, 0)),
            pl.BlockSpec((IN_DIM, HID_DIM), lambda i: (0, 0)),
            pl.BlockSpec((1, HID_DIM), lambda i: (0, 0)),
            pl.BlockSpec((HID_DIM, OUT_DIM), lambda i: (0, 0)),
        ],
        out_specs=pl.BlockSpec((RB, OUT_DIM), lambda i: (i, 0)),
        out_shape=jax.ShapeDtypeStruct((NODES, OUT_DIM), jnp.float32),
    )(p, y0, degp, W1, b1, W2)


def _final_call(q, y1, degp, b2, batch3):
    def body(q_ref, y1_ref, degp_ref, b2_ref, batch_ref, out_ref, sums, cnts):
        i = pl.program_id(0)

        @pl.when(i == 0)
        def _():
            sums[...] = jnp.zeros_like(sums)
            cnts[...] = jnp.zeros_like(cnts)

        dinv = _dinv_block(degp_ref[...])
        qv = q_ref[...]
        t1 = dinv * (qv[0] + qv[1] + y1_ref[...])
        h2 = jnp.maximum(t1 + b2_ref[...], 0.0)
        b = batch_ref[...].reshape(1, RB)
        iota = lax.broadcasted_iota(jnp.int32, (GRAPHS, RB), 0)
        onehot = (jnp.broadcast_to(b, (GRAPHS, RB)) == iota).astype(jnp.float32)
        sums[...] += jnp.dot(onehot, h2, preferred_element_type=jnp.float32)
        cnts[...] += jnp.broadcast_to(
            jnp.sum(onehot, axis=1, keepdims=True), (GRAPHS, OUT_DIM)
        )

        @pl.when(i == GRID - 1)
        def _():
            out_ref[...] = sums[...] / jnp.maximum(cnts[...], 1.0)

    return pl.pallas_call(
        body,
        grid=(GRID,),
        in_specs=[
            pl.BlockSpec((NC, RB, IN_DIM), lambda i: (0, i, 0)),
            pl.BlockSpec((RB, OUT_DIM), lambda i: (i, 0)),
            pl.BlockSpec((NC, RB, IN_DIM), lambda i: (0, i, 0)),
            pl.BlockSpec((1, OUT_DIM), lambda i: (0, 0)),
            pl.BlockSpec((1, 1, RB), lambda i: (i, 0, 0)),
        ],
        out_specs=pl.BlockSpec((GRAPHS, OUT_DIM), lambda i: (0, 0)),
        out_shape=jax.ShapeDtypeStruct((GRAPHS, OUT_DIM), jnp.float32),
        scratch_shapes=[
            pltpu.VMEM((GRAPHS, OUT_DIM), jnp.float32),
            pltpu.VMEM((GRAPHS, OUT_DIM), jnp.float32),
        ],
    )(q, y1, degp, b2, batch3)


def kernel(x, edge_index, batch, W1, b1, W2, b2):
    e = edge_index.shape[1]
    pads = jnp.stack([
        jnp.arange(EPAD - e, dtype=jnp.int32),
        jnp.full((EPAD - e,), NODES, jnp.int32),
    ])
    edges = jnp.concatenate([edge_index, pads], axis=1).reshape(2, EPAD // ACK, ACK)
    ones128 = jnp.ones((ACK, IN_DIM), jnp.float32)
    zerosf = jnp.zeros((NPAD, IN_DIM), jnp.float32)

    degp = _deg_call(edges, ones128, zerosf)
    y0 = _scale_call(x, degp)
    p = _agg_call(y0, edges, zerosf)
    y1 = _mid_call(p, y0, degp, W1, b1.reshape(1, HID_DIM), W2)
    q = _agg_call(y1, edges, zerosf)
    return _final_call(q, y1, degp, b2.reshape(1, OUT_DIM), batch.reshape(GRID, 1, RB))
